# baseline (speedup 1.0000x reference)
import jax
import jax.numpy as jnp
from jax import lax
from jax.experimental import pallas as pl
from jax.experimental.pallas import tpu as pltpu
from jax.experimental.pallas import tpu_sc as plsc

_BATCH = 16384
_NW = 32
_CHUNK = _BATCH // _NW
_GSZ = 128
_NG = _CHUNK // _GSZ
_GROUPROWS = _GSZ // 16
_V6 = 750000
_V2 = 250000


def _rsqrt(x):
    i = plsc.bitcast(x, jnp.int32)
    i = jnp.int32(0x5F3759DF) - lax.shift_right_logical(i, 1)
    y = plsc.bitcast(i, jnp.float32)
    for _ in range(3):
        y = y * (jnp.float32(1.5) - jnp.float32(0.5) * x * y * y)
    return y


def _sc_pose_kernel(idx_hbm, s2s2_hbm, trans_hbm, r9_hbm, t_hbm,
                    idx_v, ja_v, jb_v, jt_v, rowsa_v, rowsb_v, rowst_v,
                    r9_v, t2_v, sem):
    wid = lax.axis_index("s") * 2 + lax.axis_index("c")
    base = wid * _CHUNK
    iota = lax.iota(jnp.int32, 16)

    for j in range(_NG):
        pltpu.sync_copy(idx_hbm.at[wid * _NG + j], idx_v)

        def build(g, carry):
            iv = idx_v[pl.ds(g * 16, 16)]
            i6 = iv * 6
            j1 = lax.shift_right_logical(i6, 3)
            ja_v[pl.ds(g * 16, 16)] = j1
            jb_v[pl.ds(g * 16, 16)] = jnp.minimum(j1 + 1, _V6 - 1)
            jt_v[pl.ds(g * 16, 16)] = lax.shift_right_logical(iv * 2, 3)
            return carry

        lax.fori_loop(0, _GROUPROWS, build, 0)

        ca = pltpu.make_async_copy(s2s2_hbm.at[ja_v], rowsa_v, sem)
        cb = pltpu.make_async_copy(s2s2_hbm.at[jb_v], rowsb_v, sem)
        ct = pltpu.make_async_copy(trans_hbm.at[jt_v], rowst_v, sem)
        ca.start(); cb.start(); ct.start()
        ca.wait(); cb.wait(); ct.wait()

        def compute(g, carry):
            row = iota + g * 16
            iv = idx_v[pl.ds(g * 16, 16)]
            off = jnp.bitwise_and(iv * 6, 7)

            def col6(c):
                m = off + c
                va = plsc.load_gather(rowsa_v, [row, jnp.minimum(m, 7)])
                vb = plsc.load_gather(
                    rowsb_v, [row, jnp.maximum(m - 8, 0)])
                return jnp.where(m < 8, va, vb)

            a1, a2, a3 = col6(0), col6(1), col6(2)
            b1, b2, b3 = col6(3), col6(4), col6(5)

            r1 = _rsqrt(a1 * a1 + a2 * a2 + a3 * a3)
            e11, e12, e13 = a1 * r1, a2 * r1, a3 * r1
            d = e11 * b1 + e12 * b2 + e13 * b3
            u1, u2, u3 = b1 - d * e11, b2 - d * e12, b3 - d * e13
            r2 = _rsqrt(u1 * u1 + u2 * u2 + u3 * u3)
            e21, e22, e23 = u1 * r2, u2 * r2, u3 * r2
            e31 = e12 * e23 - e13 * e22
            e32 = e13 * e21 - e11 * e23
            e33 = e11 * e22 - e12 * e21

            vals = (e11, e12, e13, e21, e22, e23, e31, e32, e33)
            for c, v in enumerate(vals):
                plsc.store_scatter(
                    r9_v, [row, jnp.full((16,), c, jnp.int32)], v)

            offt = jnp.bitwise_and(iv * 2, 7)
            for c in range(2):
                tv = plsc.load_gather(rowst_v, [row, offt + c])
                plsc.store_scatter(
                    t2_v, [row, jnp.full((16,), c, jnp.int32)], tv)
            return carry

        lax.fori_loop(0, _GROUPROWS, compute, 0)

        pltpu.sync_copy(r9_v, r9_hbm.at[pl.ds(base + j * _GSZ, _GSZ)])
        pltpu.sync_copy(t2_v, t_hbm.at[pl.ds(base + j * _GSZ, _GSZ)])


@jax.jit
def _pose_lookup(tilt_index, table_s2s2, table_trans):
    idx2d = tilt_index.reshape(_NW * _NG, _GSZ)
    s2s2_8 = table_s2s2.reshape(_V6, 8)
    trans_8 = table_trans.reshape(_V2, 8)
    mesh = plsc.VectorSubcoreMesh(core_axis_name="c", subcore_axis_name="s")
    r9, t = pl.kernel(
        _sc_pose_kernel,
        mesh=mesh,
        out_type=[
            jax.ShapeDtypeStruct((_BATCH, 9), jnp.float32),
            jax.ShapeDtypeStruct((_BATCH, 2), jnp.float32),
        ],
        scratch_types=[
            pltpu.VMEM((_GSZ,), jnp.int32),
            pltpu.VMEM((_GSZ,), jnp.int32),
            pltpu.VMEM((_GSZ,), jnp.int32),
            pltpu.VMEM((_GSZ,), jnp.int32),
            pltpu.VMEM((_GSZ, 8), jnp.float32),
            pltpu.VMEM((_GSZ, 8), jnp.float32),
            pltpu.VMEM((_GSZ, 8), jnp.float32),
            pltpu.VMEM((_GSZ, 9), jnp.float32),
            pltpu.VMEM((_GSZ, 2), jnp.float32),
            pltpu.SemaphoreType.DMA,
        ],
        compiler_params=pltpu.CompilerParams(
            needs_layout_passes=False, use_tc_tiling_on_sc=False),
    )(idx2d, s2s2_8, trans_8)
    return r9, t


def kernel(tilt_index, y, table_s2s2, table_trans):
    r9, t = _pose_lookup(tilt_index, table_s2s2, table_trans)
    R = r9.reshape(_BATCH, 3, 3)
    return (R, t)


# speedup vs baseline: 3.2942x; 3.2942x over previous
import jax
import jax.numpy as jnp
from jax import lax
from jax.experimental import pallas as pl
from jax.experimental.pallas import tpu as pltpu
from jax.experimental.pallas import tpu_sc as plsc

_BATCH = 16384
_NW = 32
_CHUNK = _BATCH // _NW
_GSZ = 128
_NG = _CHUNK // _GSZ
_GROUPROWS = _GSZ // 16
_CSTRIDE = 125000


def _rsqrt(x):
    i = plsc.bitcast(x, jnp.int32)
    i = jnp.int32(0x5F3759DF) - lax.shift_right_logical(i, 1)
    y = plsc.bitcast(i, jnp.float32)
    for _ in range(3):
        y = y * (jnp.float32(1.5) - jnp.float32(0.5) * x * y * y)
    return y


def _sc_pose_kernel(idx_hbm, s6_hbm, s2_hbm, r9_hbm, t_hbm, *scratch):
    idx_v = scratch[0]
    jrow_refs = scratch[1:9]
    rows_refs = scratch[9:17]
    r9_v, t2_v, sem = scratch[17], scratch[18], scratch[19]

    wid = lax.axis_index("s") * 2 + lax.axis_index("c")
    base = wid * _CHUNK
    iota = lax.iota(jnp.int32, 16)
    del iota

    for j in range(_NG):
        pltpu.sync_copy(idx_hbm.at[wid * _NG + j], idx_v)

        def build(g, carry):
            iv = idx_v[pl.ds(g * 16, 16)]
            t = lax.shift_right_logical(iv, 3)
            for c in range(8):
                jrow_refs[c][pl.ds(g * 16, 16)] = t + (c % 6 if c < 6 else c - 6) * _CSTRIDE
            return carry

        lax.fori_loop(0, _GROUPROWS, build, 0)

        copies = []
        for c in range(6):
            copies.append(pltpu.make_async_copy(
                s6_hbm.at[jrow_refs[c]], rows_refs[c], sem))
        for c in range(2):
            copies.append(pltpu.make_async_copy(
                s2_hbm.at[jrow_refs[6 + c]], rows_refs[6 + c], sem))
        for cp in copies:
            cp.start()
        for cp in copies:
            cp.wait()

        def compute(g, carry):
            s = g * 16
            row = lax.iota(jnp.int32, 16) + s
            iv = idx_v[pl.ds(s, 16)]
            off = jnp.bitwise_and(iv, 7)

            def col(c):
                return plsc.load_gather(rows_refs[c], [row, off])

            a1, a2, a3 = col(0), col(1), col(2)
            b1, b2, b3 = col(3), col(4), col(5)

            r1 = _rsqrt(a1 * a1 + a2 * a2 + a3 * a3)
            e11, e12, e13 = a1 * r1, a2 * r1, a3 * r1
            d = e11 * b1 + e12 * b2 + e13 * b3
            u1, u2, u3 = b1 - d * e11, b2 - d * e12, b3 - d * e13
            r2 = _rsqrt(u1 * u1 + u2 * u2 + u3 * u3)
            e21, e22, e23 = u1 * r2, u2 * r2, u3 * r2
            e31 = e12 * e23 - e13 * e22
            e32 = e13 * e21 - e11 * e23
            e33 = e11 * e22 - e12 * e21

            vals = (e11, e12, e13, e21, e22, e23, e31, e32, e33)
            for c, v in enumerate(vals):
                plsc.store_scatter(
                    r9_v, [row, jnp.full((16,), c, jnp.int32)], v)
            for c in range(2):
                tv = col(6 + c)
                plsc.store_scatter(
                    t2_v, [row, jnp.full((16,), c, jnp.int32)], tv)
            return carry

        lax.fori_loop(0, _GROUPROWS, compute, 0)

        pltpu.sync_copy(r9_v, r9_hbm.at[pl.ds(base + j * _GSZ, _GSZ)])
        pltpu.sync_copy(t2_v, t_hbm.at[pl.ds(base + j * _GSZ, _GSZ)])


@jax.jit
def _pose_lookup(tilt_index, table_s2s2, table_trans):
    idx2d = tilt_index.reshape(_NW * _NG, _GSZ)
    s6 = table_s2s2.T.reshape(6 * _CSTRIDE, 8)
    s2 = table_trans.T.reshape(2 * _CSTRIDE, 8)
    mesh = plsc.VectorSubcoreMesh(core_axis_name="c", subcore_axis_name="s")
    r9, t = pl.kernel(
        _sc_pose_kernel,
        mesh=mesh,
        out_type=[
            jax.ShapeDtypeStruct((_BATCH, 9), jnp.float32),
            jax.ShapeDtypeStruct((_BATCH, 2), jnp.float32),
        ],
        scratch_types=(
            [pltpu.VMEM((_GSZ,), jnp.int32)]
            + [pltpu.VMEM((_GSZ,), jnp.int32) for _ in range(8)]
            + [pltpu.VMEM((_GSZ, 8), jnp.float32) for _ in range(8)]
            + [pltpu.VMEM((_GSZ, 9), jnp.float32),
               pltpu.VMEM((_GSZ, 2), jnp.float32),
               pltpu.SemaphoreType.DMA]
        ),
        compiler_params=pltpu.CompilerParams(
            needs_layout_passes=False, use_tc_tiling_on_sc=False),
    )(idx2d, s6, s2)
    return r9, t


def kernel(tilt_index, y, table_s2s2, table_trans):
    r9, t = _pose_lookup(tilt_index, table_s2s2, table_trans)
    R = r9.reshape(_BATCH, 3, 3)
    return (R, t)


# speedup vs baseline: 3.3082x; 1.0042x over previous
import jax
import jax.numpy as jnp
from jax import lax
from jax.experimental import pallas as pl
from jax.experimental.pallas import tpu as pltpu
from jax.experimental.pallas import tpu_sc as plsc

_BATCH = 16384
_NW = 32
_CHUNK = _BATCH // _NW
_GSZ = 128
_NG = _CHUNK // _GSZ
_GROUPROWS = _GSZ // 16
_CSTRIDE = 125000


def _rsqrt(x):
    i = plsc.bitcast(x, jnp.int32)
    i = jnp.int32(0x5F3759DF) - lax.shift_right_logical(i, 1)
    y = plsc.bitcast(i, jnp.float32)
    for _ in range(3):
        y = y * (jnp.float32(1.5) - jnp.float32(0.5) * x * y * y)
    return y


def _sc_pose_kernel(idx_hbm, s6_hbm, s2_hbm, r9_hbm, t_hbm, *scratch):
    idx_v = scratch[0]
    jrow_refs = scratch[1:9]
    rows_refs = scratch[9:17]
    r9_v, t2_v, sem = scratch[17], scratch[18], scratch[19]

    wid = lax.axis_index("s") * 2 + lax.axis_index("c")
    base = wid * _CHUNK

    for j in range(_NG):
        pltpu.sync_copy(idx_hbm.at[wid * _NG + j], idx_v)

        def build(g, carry):
            iv = idx_v[pl.ds(g * 16, 16)]
            t = lax.shift_right_logical(iv, 3)
            for c in range(8):
                col_in_table = c if c < 6 else c - 6
                jrow_refs[c][pl.ds(g * 16, 16)] = t + col_in_table * _CSTRIDE
            return carry

        lax.fori_loop(0, _GROUPROWS, build, 0)

        copies = []
        for c in range(6):
            copies.append(pltpu.make_async_copy(
                s6_hbm.at[jrow_refs[c]], rows_refs[c], sem))
        for c in range(2):
            copies.append(pltpu.make_async_copy(
                s2_hbm.at[jrow_refs[6 + c]], rows_refs[6 + c], sem))
        for cp in copies:
            cp.start()
        for cp in copies:
            cp.wait()

        def compute(g, carry):
            s = g * 16
            row = lax.iota(jnp.int32, 16) + s
            iv = idx_v[pl.ds(s, 16)]
            off = jnp.bitwise_and(iv, 7)

            def col(c):
                return plsc.load_gather(rows_refs[c], [row, off])

            a1, a2, a3 = col(0), col(1), col(2)
            b1, b2, b3 = col(3), col(4), col(5)

            r1 = _rsqrt(a1 * a1 + a2 * a2 + a3 * a3)
            e11, e12, e13 = a1 * r1, a2 * r1, a3 * r1
            d = e11 * b1 + e12 * b2 + e13 * b3
            u1, u2, u3 = b1 - d * e11, b2 - d * e12, b3 - d * e13
            r2 = _rsqrt(u1 * u1 + u2 * u2 + u3 * u3)
            e21, e22, e23 = u1 * r2, u2 * r2, u3 * r2
            e31 = e12 * e23 - e13 * e22
            e32 = e13 * e21 - e11 * e23
            e33 = e11 * e22 - e12 * e21

            vals = (e11, e12, e13, e21, e22, e23, e31, e32, e33)
            for c, v in enumerate(vals):
                plsc.store_scatter(
                    r9_v, [row, jnp.full((16,), c, jnp.int32)], v)
            for c in range(2):
                tv = col(6 + c)
                plsc.store_scatter(
                    t2_v, [row, jnp.full((16,), c, jnp.int32)], tv)
            return carry

        lax.fori_loop(0, _GROUPROWS, compute, 0)

        pltpu.sync_copy(r9_v, r9_hbm.at[pl.ds(base + j * _GSZ, _GSZ)])
        pltpu.sync_copy(t2_v, t_hbm.at[pl.ds(base + j * _GSZ, _GSZ)])


@jax.jit
def _pose_lookup(tilt_index, table_s2s2, table_trans):
    idx2d = tilt_index.reshape(_NW * _NG, _GSZ)
    s6 = table_s2s2.T.reshape(6 * _CSTRIDE, 8)
    s2 = table_trans.T.reshape(2 * _CSTRIDE, 8)
    mesh = plsc.VectorSubcoreMesh(core_axis_name="c", subcore_axis_name="s")
    r9, t = pl.kernel(
        _sc_pose_kernel,
        mesh=mesh,
        out_type=[
            jax.ShapeDtypeStruct((_BATCH, 9), jnp.float32),
            jax.ShapeDtypeStruct((_BATCH, 2), jnp.float32),
        ],
        scratch_types=(
            [pltpu.VMEM((_GSZ,), jnp.int32)]
            + [pltpu.VMEM((_GSZ,), jnp.int32) for _ in range(8)]
            + [pltpu.VMEM((_GSZ, 8), jnp.float32) for _ in range(8)]
            + [pltpu.VMEM((_GSZ, 9), jnp.float32),
               pltpu.VMEM((_GSZ, 2), jnp.float32),
               pltpu.SemaphoreType.DMA]
        ),
        compiler_params=pltpu.CompilerParams(
            needs_layout_passes=False, use_tc_tiling_on_sc=False),
    )(idx2d, s6, s2)
    return r9, t


def kernel(tilt_index, y, table_s2s2, table_trans):
    r9, t = _pose_lookup(tilt_index, table_s2s2, table_trans)
    R = r9.reshape(_BATCH, 3, 3)
    return (R, t)
